# baseline (speedup 1.0000x reference)
import functools

import jax
import jax.numpy as jnp
from jax import lax
from jax.experimental import pallas as pl
from jax.experimental.pallas import tpu as pltpu
from jax.experimental.pallas import tpu_sc as plsc

_D = 64
_CH = 128
_BS = 4
_NB = 2
_NW = 32


@functools.lru_cache(maxsize=None)
def _build(n_b, n_t):
    b_per_w = n_b // _NW
    n_super = b_per_w // _BS
    assert n_super % _NB == 0 and n_super >= 2 * _NB
    chunks = []
    for b in range(_BS):
        o = 0
        while o < n_t:
            l = min(_CH, n_t - o)
            chunks.append((b, o, l))
            o += l
    mesh = plsc.VectorSubcoreMesh(core_axis_name="c", subcore_axis_name="s")

    @functools.partial(
        pl.kernel,
        out_type=jax.ShapeDtypeStruct((n_b, n_t, 128), jnp.float32),
        mesh=mesh,
        scratch_types=[
            pltpu.VMEM((b_per_w, n_t), jnp.int32),
            [pltpu.VMEM((_BS, n_t, _D), jnp.float32) for _ in range(_NB)],
            [pltpu.SemaphoreType.DMA for _ in range(_NB)],
            [pltpu.SemaphoreType.DMA for _ in range(_NB)],
        ],
        compiler_params=pltpu.CompilerParams(use_tc_tiling_on_sc=False),
    )
    def gather_kernel(idx_hbm, table_hbm, out_hbm, idx_v, bufs, gsems, ssems):
        wid = lax.axis_index("s") * 2 + lax.axis_index("c")
        b_base = wid * b_per_w
        pltpu.sync_copy(idx_hbm.at[pl.ds(b_base, b_per_w)], idx_v)

        def out_slice(s):
            return out_hbm.at[pl.ds(b_base + s * _BS, _BS), :, pl.ds(0, _D)]

        def issue(s, k):
            for (b, o, l) in chunks:
                pltpu.async_copy(
                    table_hbm.at[idx_v.at[s * _BS + b, pl.ds(o, l)]],
                    bufs[k].at[b, pl.ds(o, l)], gsems[k])

        def wait_gathers(s, k):
            for (b, o, l) in chunks:
                pltpu.make_async_copy(
                    table_hbm.at[idx_v.at[s * _BS + b, pl.ds(o, l)]],
                    bufs[k].at[b, pl.ds(o, l)], gsems[k]).wait()

        def start_store(s, k):
            pltpu.async_copy(bufs[k], out_slice(s), ssems[k])

        def wait_store(s, k):
            pltpu.make_async_copy(bufs[k], out_slice(s), ssems[k]).wait()

        def drain(s, k):
            wait_gathers(s, k)
            start_store(s, k)
            wait_store(s, k)

        for j in range(_NB - 1):
            issue(j, j)

        @pl.loop(0, n_super - _NB, step=_NB)
        def _(s):
            for j in range(_NB):
                issue(s + j + _NB - 1, (j + _NB - 1) % _NB)
                drain(s + j, j)

        issue(n_super - 1, (_NB - 1) % _NB)
        for j in range(_NB):
            drain(n_super - _NB + j, j)

    return gather_kernel


def kernel(positions, encoding_matrix):
    n_b, n_t = positions.shape
    h = n_b // 2
    gk = _build(h, n_t)
    o1 = gk(positions[:h], encoding_matrix)
    o2 = gk(positions[h:], encoding_matrix)
    return jnp.concatenate([o1[:, :, :_D], o2[:, :, :_D]], axis=0)


# speedup vs baseline: 1.6638x; 1.6638x over previous
import functools

import jax
import jax.numpy as jnp
from jax import lax
from jax.experimental import pallas as pl
from jax.experimental.pallas import tpu as pltpu
from jax.experimental.pallas import tpu_sc as plsc

_D = 64
_CH = 128
_BS = 4
_NB = 2
_NW = 32


@functools.lru_cache(maxsize=None)
def _build(n_b, n_t):
    b_per_w = n_b // _NW
    n_super = b_per_w // _BS
    assert n_super % _NB == 0 and n_super >= 2 * _NB
    chunks = []
    for b in range(_BS):
        o = 0
        while o < n_t:
            l = min(_CH, n_t - o)
            chunks.append((b, o, l))
            o += l
    mesh = plsc.VectorSubcoreMesh(core_axis_name="c", subcore_axis_name="s")

    @functools.partial(
        pl.kernel,
        out_type=jax.ShapeDtypeStruct((n_b, n_t, 128), jnp.float32),
        mesh=mesh,
        scratch_types=[
            pltpu.VMEM((b_per_w, n_t), jnp.int32),
            [pltpu.VMEM((_BS, n_t, _D), jnp.float32) for _ in range(_NB)],
            [pltpu.SemaphoreType.DMA for _ in range(_NB)],
            [pltpu.SemaphoreType.DMA for _ in range(_NB)],
        ],
        compiler_params=pltpu.CompilerParams(use_tc_tiling_on_sc=False),
    )
    def gather_kernel(idx_hbm, table_hbm, out_hbm, idx_v, bufs, gsems, ssems):
        wid = lax.axis_index("s") * 2 + lax.axis_index("c")
        b_base = wid * b_per_w
        pltpu.sync_copy(idx_hbm.at[pl.ds(b_base, b_per_w)], idx_v)

        def out_slice(s):
            return out_hbm.at[pl.ds(b_base + s * _BS, _BS), :, pl.ds(0, _D)]

        def issue(s, k):
            for (b, o, l) in chunks:
                pltpu.async_copy(
                    table_hbm.at[idx_v.at[s * _BS + b, pl.ds(o, l)]],
                    bufs[k].at[b, pl.ds(o, l)], gsems[k])

        def wait_gathers(s, k):
            for (b, o, l) in chunks:
                pltpu.make_async_copy(
                    table_hbm.at[idx_v.at[s * _BS + b, pl.ds(o, l)]],
                    bufs[k].at[b, pl.ds(o, l)], gsems[k]).wait()

        def start_store(s, k):
            pltpu.async_copy(bufs[k], out_slice(s), ssems[k])

        def wait_store(s, k):
            pltpu.make_async_copy(bufs[k], out_slice(s), ssems[k]).wait()

        def drain(s, k):
            wait_gathers(s, k)
            start_store(s, k)
            wait_store(s, k)

        for j in range(_NB - 1):
            issue(j, j)

        @pl.loop(0, n_super - _NB, step=_NB)
        def _(s):
            for j in range(_NB):
                issue(s + j + _NB - 1, (j + _NB - 1) % _NB)
                drain(s + j, j)

        issue(n_super - 1, (_NB - 1) % _NB)
        for j in range(_NB):
            drain(n_super - _NB + j, j)

    return gather_kernel


def kernel(positions, encoding_matrix):
    n_b, n_t = positions.shape
    out128 = _build(n_b, n_t)(positions, encoding_matrix)
    return out128[:, :, :_D]
